# Initial kernel scaffold; baseline (speedup 1.0000x reference)
import jax
import jax.numpy as jnp
from jax.experimental import pallas as pl


def kernel(x, w0, b0, wa0, wa1, wa2, ba0, ba1, ba2, wp, bp, wproj, bproj):
    raise NotImplementedError("write your pallas kernel here")


# speedup vs baseline: 1.2473x; 1.2473x over previous
import functools

import jax
import jax.numpy as jnp
from jax.experimental import pallas as pl
from jax.experimental.pallas import tpu as pltpu


def _aspp_kernel(x_ref, w0_ref, wa_ref, wpjm_ref, wpp_ref, wp_ref,
                 b0_ref, ba_ref, bp_ref, bpj_ref, o_ref, slab,
                 *, H, W, R, TH, rates):
    Cin = x_ref.shape[-1]
    Cout = o_ref.shape[-1]
    Hp = H + 2 * R
    Wp = W + 2 * R

    x = x_ref[0]

    s = jnp.sum(x.astype(jnp.float32), axis=(0, 1)).reshape(1, Cin)
    mean = (s * (1.0 / float(H * W))).astype(jnp.bfloat16)
    py = jnp.maximum(
        jnp.dot(mean, wp_ref[...], preferred_element_type=jnp.float32)
        + bp_ref[...], 0.0)
    pool_c = jnp.dot(py.astype(jnp.bfloat16), wpp_ref[...],
                     preferred_element_type=jnp.float32)

    slab[R:R + H, R:R + W, :] = x
    slab[0:R, :, :] = jnp.zeros((R, Wp, Cin), slab.dtype)
    slab[R + H:Hp, :, :] = jnp.zeros((R, Wp, Cin), slab.dtype)
    slab[R:R + H, 0:R, :] = jnp.zeros((H, R, Cin), slab.dtype)
    slab[R:R + H, R + W:Wp, :] = jnp.zeros((H, R, Cin), slab.dtype)

    M = TH * W
    for t in range(H // TH):
        base = t * TH
        xc = x[base:base + TH].reshape(M, Cin)
        y0 = jnp.maximum(
            jnp.dot(xc, w0_ref[...], preferred_element_type=jnp.float32)
            + b0_ref[...], 0.0)
        ys = [y0.astype(jnp.bfloat16)]

        for bi, r in enumerate(rates):
            parts = []
            for ky in range(3):
                r0 = R + base + (ky - 1) * r
                for kx in range(3):
                    c0 = R + (kx - 1) * r
                    parts.append(slab[r0:r0 + TH, c0:c0 + W, :].reshape(M, Cin))
            patch = jnp.concatenate(parts, axis=1)
            yb = jnp.maximum(
                jnp.dot(patch, wa_ref[bi], preferred_element_type=jnp.float32)
                + ba_ref[bi], 0.0)
            ys.append(yb.astype(jnp.bfloat16))

        ycat = jnp.concatenate(ys, axis=1)
        o = jnp.maximum(
            jnp.dot(ycat, wpjm_ref[...], preferred_element_type=jnp.float32)
            + bpj_ref[...] + pool_c, 0.0)
        o_ref[0, base:base + TH] = o.reshape(TH, W, Cout)


def kernel(x, w0, b0, wa0, wa1, wa2, ba0, ba1, ba2, wp, bp, wproj, bproj):
    rates = (1, 2, 3)
    R = max(rates)
    N, Cin, H, W = x.shape
    Cout = w0.shape[-1]
    TH = 16 if H % 16 == 0 else H
    bf16 = jnp.bfloat16

    xt = jnp.transpose(x, (0, 2, 3, 1)).astype(bf16)
    wa = jnp.stack([wa0.reshape(9 * Cin, Cout),
                    wa1.reshape(9 * Cin, Cout),
                    wa2.reshape(9 * Cin, Cout)]).astype(bf16)
    ba = jnp.stack([ba0, ba1, ba2])
    wpjm = wproj[:(1 + len(rates)) * Cout].astype(bf16)
    wpp = wproj[(1 + len(rates)) * Cout:].astype(bf16)

    kern = functools.partial(_aspp_kernel, H=H, W=W, R=R, TH=TH, rates=rates)
    out_nhwc = pl.pallas_call(
        kern,
        out_shape=jax.ShapeDtypeStruct((N, H, W, Cout), jnp.float32),
        grid_spec=pltpu.PrefetchScalarGridSpec(
            num_scalar_prefetch=0,
            grid=(N,),
            in_specs=[
                pl.BlockSpec((1, H, W, Cin), lambda n: (n, 0, 0, 0)),
                pl.BlockSpec((Cin, Cout), lambda n: (0, 0)),
                pl.BlockSpec((3, 9 * Cin, Cout), lambda n: (0, 0, 0)),
                pl.BlockSpec(((1 + len(rates)) * Cout, Cout), lambda n: (0, 0)),
                pl.BlockSpec((Cout, Cout), lambda n: (0, 0)),
                pl.BlockSpec((Cin, Cout), lambda n: (0, 0)),
                pl.BlockSpec((1, Cout), lambda n: (0, 0)),
                pl.BlockSpec((3, 1, Cout), lambda n: (0, 0, 0)),
                pl.BlockSpec((1, Cout), lambda n: (0, 0)),
                pl.BlockSpec((1, Cout), lambda n: (0, 0)),
            ],
            out_specs=pl.BlockSpec((1, H, W, Cout), lambda n: (n, 0, 0, 0)),
            scratch_shapes=[pltpu.VMEM((H + 2 * R, W + 2 * R, Cin), bf16)],
        ),
        compiler_params=pltpu.CompilerParams(
            dimension_semantics=("parallel",),
            vmem_limit_bytes=64 * 1024 * 1024,
        ),
    )(xt, w0.astype(bf16), wa, wpjm, wpp, wp.astype(bf16),
      b0, ba, bp, bproj)

    return jnp.transpose(out_nhwc, (0, 3, 1, 2))


# speedup vs baseline: 1.5141x; 1.2139x over previous
import functools

import jax
import jax.numpy as jnp
from jax.experimental import pallas as pl
from jax.experimental.pallas import tpu as pltpu


def _aspp_kernel(x_ref, w0_ref, wa_ref, wpjm_ref, wpp_ref, wp_ref,
                 b0_ref, ba_ref, bp_ref, bpj_ref, o_ref, sslab, ystash,
                 *, H, W, R, TH, rates):
    Cin = x_ref.shape[-1]
    Cout = o_ref.shape[-1]
    Hp = H + 2 * R

    x = x_ref[0]

    for d in range(2 * R + 1):
        wlo = max(0, R - d)
        whi = min(W, W + R - d)
        slo = wlo + d - R
        shi = whi + d - R
        sslab[d, R:R + H, wlo:whi, :] = x[:, slo:shi, :]
        if wlo > 0:
            sslab[d, R:R + H, 0:wlo, :] = jnp.zeros((H, wlo, Cin), x.dtype)
        if whi < W:
            sslab[d, R:R + H, whi:W, :] = jnp.zeros((H, W - whi, Cin), x.dtype)
        sslab[d, 0:R] = jnp.zeros((R, W, Cin), x.dtype)
        sslab[d, R + H:Hp] = jnp.zeros((R, W, Cin), x.dtype)

    M = TH * W
    n_tiles = H // TH

    s_parts = []
    for t in range(n_tiles):
        base = t * TH
        xc = x[base:base + TH].reshape(M, Cin)
        s_parts.append(jnp.sum(xc.astype(jnp.float32), axis=0))
        y0 = jnp.maximum(
            jnp.dot(xc, w0_ref[...], preferred_element_type=jnp.float32)
            + b0_ref[...], 0.0)
        ys = [y0.astype(jnp.bfloat16)]

        for bi, r in enumerate(rates):
            yb = ba_ref[bi]
            for ky in range(3):
                r0 = R + base + (ky - 1) * r
                for kx in range(3):
                    d = R + (kx - 1) * r
                    patch = sslab[d, r0:r0 + TH, :, :].reshape(M, Cin)
                    yb = yb + jnp.dot(patch, wa_ref[bi * 9 + ky * 3 + kx],
                                      preferred_element_type=jnp.float32)
            yb = jnp.maximum(yb, 0.0)
            ys.append(yb.astype(jnp.bfloat16))
        ystash[t] = jnp.concatenate(ys, axis=1)

    s = sum(s_parts).reshape(1, Cin)
    mean = (s * (1.0 / float(H * W))).astype(jnp.bfloat16)
    py = jnp.maximum(
        jnp.dot(mean, wp_ref[...], preferred_element_type=jnp.float32)
        + bp_ref[...], 0.0)
    pool_c = jnp.dot(py.astype(jnp.bfloat16), wpp_ref[...],
                     preferred_element_type=jnp.float32)

    for t in range(n_tiles):
        base = t * TH
        o = jnp.maximum(
            jnp.dot(ystash[t], wpjm_ref[...], preferred_element_type=jnp.float32)
            + bpj_ref[...] + pool_c, 0.0)
        o_ref[0, base:base + TH] = o.reshape(TH, W, Cout)


def kernel(x, w0, b0, wa0, wa1, wa2, ba0, ba1, ba2, wp, bp, wproj, bproj):
    rates = (1, 2, 3)
    R = max(rates)
    N, Cin, H, W = x.shape
    Cout = w0.shape[-1]
    TH = 16 if H % 16 == 0 else H
    bf16 = jnp.bfloat16

    xt = jnp.transpose(x, (0, 2, 3, 1)).astype(bf16)
    wa = jnp.concatenate([wa0.reshape(9, Cin, Cout),
                          wa1.reshape(9, Cin, Cout),
                          wa2.reshape(9, Cin, Cout)]).astype(bf16)
    ba = jnp.stack([ba0, ba1, ba2])
    wpjm = wproj[:(1 + len(rates)) * Cout].astype(bf16)
    wpp = wproj[(1 + len(rates)) * Cout:].astype(bf16)

    kern = functools.partial(_aspp_kernel, H=H, W=W, R=R, TH=TH, rates=rates)
    out_nhwc = pl.pallas_call(
        kern,
        out_shape=jax.ShapeDtypeStruct((N, H, W, Cout), jnp.float32),
        grid_spec=pltpu.PrefetchScalarGridSpec(
            num_scalar_prefetch=0,
            grid=(N,),
            in_specs=[
                pl.BlockSpec((1, H, W, Cin), lambda n: (n, 0, 0, 0)),
                pl.BlockSpec((Cin, Cout), lambda n: (0, 0)),
                pl.BlockSpec((27, Cin, Cout), lambda n: (0, 0, 0)),
                pl.BlockSpec(((1 + len(rates)) * Cout, Cout), lambda n: (0, 0)),
                pl.BlockSpec((Cout, Cout), lambda n: (0, 0)),
                pl.BlockSpec((Cin, Cout), lambda n: (0, 0)),
                pl.BlockSpec((1, Cout), lambda n: (0, 0)),
                pl.BlockSpec((3, 1, Cout), lambda n: (0, 0, 0)),
                pl.BlockSpec((1, Cout), lambda n: (0, 0)),
                pl.BlockSpec((1, Cout), lambda n: (0, 0)),
            ],
            out_specs=pl.BlockSpec((1, H, W, Cout), lambda n: (n, 0, 0, 0)),
            scratch_shapes=[
                pltpu.VMEM((2 * R + 1, H + 2 * R, W, Cin), bf16),
                pltpu.VMEM((H // TH, TH * W, (1 + len(rates)) * Cout), bf16),
            ],
        ),
        compiler_params=pltpu.CompilerParams(
            dimension_semantics=("parallel",),
            vmem_limit_bytes=64 * 1024 * 1024,
        ),
    )(xt, w0.astype(bf16), wa, wpjm, wpp, wp.astype(bf16),
      b0, ba, bp, bproj)

    return jnp.transpose(out_nhwc, (0, 3, 1, 2))
